# baseline (speedup 1.0000x reference)
import functools
import jax
import jax.numpy as jnp
from jax import lax
from jax.experimental import pallas as pl
from jax.experimental.pallas import tpu as pltpu
from jax.experimental.pallas import tpu_sc as plsc

_MODEL_DIM = 64
_BATCH = 4096
_SEQ = 200
_VOCAB = 1000000

_info = plsc.get_sparse_core_info()
_NC = _info.num_cores
_NS = _info.num_subcores
_SPC = _SEQ // _NC
_BBLK = _BATCH // _NS
_IBLK = 128
_NG = _BBLK // _IBLK
_TPAD = _BBLK + 1
_DT = _MODEL_DIM // 8
_BT = _BATCH // 128

_mesh = plsc.VectorSubcoreMesh(core_axis_name="c", subcore_axis_name="s")


@functools.partial(
    pl.kernel,
    mesh=_mesh,
    out_type=jax.ShapeDtypeStruct((_SEQ, _DT, _BT, 8, 128), jnp.float32),
    scratch_types=[
        pltpu.VMEM((_SPC, _NG, _IBLK), jnp.int32),
        pltpu.VMEM((_BBLK, _MODEL_DIM), jnp.float32),
        pltpu.VMEM((_BBLK, _MODEL_DIM), jnp.float32),
        pltpu.VMEM((_MODEL_DIM, _TPAD), jnp.float32),
        pltpu.VMEM((_MODEL_DIM, _TPAD), jnp.float32),
        pltpu.SemaphoreType.DMA,
        pltpu.SemaphoreType.DMA,
        pltpu.SemaphoreType.DMA,
        pltpu.SemaphoreType.DMA,
    ],
    compiler_params=pltpu.CompilerParams(
        use_tc_tiling_on_sc=False, needs_layout_passes=False
    ),
)
def _emb_lookup(
    ids_hbm, table_hbm, out_hbm,
    ids_v, rin0, rin1, rout0, rout1, gsem0, gsem1, osem0, osem1,
):
    rin = [rin0, rin1]
    rout = [rout0, rout1]
    gsem = [gsem0, gsem1]
    osem = [osem0, osem1]
    cid = lax.axis_index("c")
    sid = lax.axis_index("s")
    s0 = cid * _SPC
    b0 = sid * _BBLK
    for j in range(_NG):
        pltpu.sync_copy(
            ids_hbm.at[pl.ds(s0, _SPC), pl.ds(b0 + j * _IBLK, _IBLK)],
            ids_v.at[:, j, :],
        )


    def gather_pos(li, buf, sem):
        for j in range(_NG):
            pltpu.async_copy(
                table_hbm.at[ids_v.at[li, j]],
                buf.at[pl.ds(j * _IBLK, _IBLK)],
                sem,
            )

    def wait_pos(li, buf, sem):
        for j in range(_NG):
            pltpu.make_async_copy(
                table_hbm.at[ids_v.at[li, j]],
                buf.at[pl.ds(j * _IBLK, _IBLK)],
                sem,
            ).wait()

    def out_chunks(li, p):
        for dt in range(_DT):
            for bt in range(_NG):
                yield (
                    rout[p].at[pl.ds(8 * dt, 8), pl.ds(128 * bt, 128)],
                    out_hbm.at[s0 + li, dt, _NG * sid + bt],
                )

    gather_pos(0, rin0, gsem0)

    lane = lax.iota(jnp.int32, 16)
    dvecs = [k * 16 + lane for k in range(_MODEL_DIM // 16)]

    def outer(g, carry):
        for p in range(2):
            li = 2 * g + p
            wait_pos(li, rin[p], gsem[p])

            @pl.when(li + 1 < _SPC)
            def _():
                gather_pos(li + 1, rin[1 - p], gsem[1 - p])

            @pl.when(li >= 2)
            def _():
                for src, dst in out_chunks(li - 2, p):
                    pltpu.make_async_copy(src, dst, osem[p]).wait()

            def tbody(r, c):
                rvec = jnp.full((16,), r, jnp.int32)
                vs = [
                    rin[p][r, pl.ds(k * 16, 16)] * 8.0
                    for k in range(_MODEL_DIM // 16)
                ]
                for k in range(_MODEL_DIM // 16):
                    plsc.store_scatter(rout[p], [dvecs[k], rvec], vs[k])
                return c

            lax.fori_loop(0, _BBLK, tbody, 0, unroll=4)

            for src, dst in out_chunks(li, p):
                pltpu.async_copy(src, dst, osem[p])
        return carry

    lax.fori_loop(0, _SPC // 2, outer, 0)

    for p, li in ((0, _SPC - 2), (1, _SPC - 1)):
        for src, dst in out_chunks(li, p):
            pltpu.make_async_copy(src, dst, osem[p]).wait()


def kernel(token_ids_batch, embeddings_table):
    ids_t = token_ids_batch.T.astype(jnp.int32)
    tbl_flat = lax.optimization_barrier(
        jnp.reshape(embeddings_table, (_MODEL_DIM * _VOCAB,))
    )
    tbl_lin = jnp.reshape(tbl_flat, (_VOCAB, _MODEL_DIM))
    out5 = _emb_lookup(ids_t, tbl_lin)
    out = jnp.transpose(out5, (2, 4, 0, 1, 3))
    return jnp.reshape(out, (_BATCH, _SEQ, _MODEL_DIM))
